# Initial kernel scaffold; baseline (speedup 1.0000x reference)
import jax
import jax.numpy as jnp
from jax.experimental import pallas as pl


def kernel(input_ids, item_codes, centroids):
    raise NotImplementedError("write your pallas kernel here")


# speedup vs baseline: 28.7770x; 28.7770x over previous
import jax
import jax.numpy as jnp
from jax import lax
from jax.experimental import pallas as pl
from jax.experimental.pallas import tpu as pltpu
from jax.experimental.pallas import tpu_sc as plsc

B, S = 1024, 200
PQ_M, SUB, CODEBOOK, EMB = 8, 16, 256, 128
N = B * S
NC, NS = 2, 16
NW = NC * NS
TPW = N // NW
BLK = 256
NBLK = TPW // BLK
CODES_BLK = BLK * PQ_M
IDX_CHUNK = 128
C_CHUNKS = CODES_BLK // IDX_CHUNK


NUM_ITEMS = 1000000
NUM_CODE_WORDS = (NUM_ITEMS + 1) * PQ_M


def _sc_body(ids_hbm, codes_hbm, cent_hbm, out_hbm,
             ids_v, fidx_v, codes_v, out_v, cent_sh, sem):
    sid = lax.axis_index("s")
    wid = sid * NC + lax.axis_index("c")

    @pl.when(sid == 0)
    def _stage_table():
        pltpu.sync_copy(cent_hbm, cent_sh)
    plsc.subcore_barrier()
    lane = lax.iota(jnp.int32, 16)
    colpat = lax.bitwise_and(lane, 7)
    rowpat = lax.shift_right_logical(lane, 3)
    mpat = colpat * CODEBOOK

    def block_body(blk, carry):
        t0 = wid * TPW + blk * BLK
        pltpu.sync_copy(ids_hbm.at[pl.ds(t0, BLK)], ids_v)

        def eidx_body(i, c2):
            ids16 = ids_v[pl.ds(i * 16, 16)]
            for p in range(8):
                toks = ids16.at[rowpat + 2 * p].get(
                    mode="promise_in_bounds")
                fidx_v[pl.ds(i * 128 + p * 16, 16)] = toks * PQ_M + colpat
            return c2
        lax.fori_loop(0, BLK // 16, eidx_body, 0)

        hs = [
            pltpu.async_copy(
                codes_hbm.at[fidx_v.at[pl.ds(c * IDX_CHUNK, IDX_CHUNK)]],
                codes_v.at[pl.ds(c * IDX_CHUNK, IDX_CHUNK)],
                sem)
            for c in range(C_CHUNKS)
        ]
        for h in hs:
            h.wait()

        def fidx_body(i, c2):
            g = codes_v[pl.ds(i * 16, 16)]
            fidx_v[pl.ds(i * 16, 16)] = g + mpat
            return c2
        lax.fori_loop(0, CODES_BLK // 16, fidx_body, 0)

        hs = [
            pltpu.async_copy(
                cent_sh.at[fidx_v.at[pl.ds(c * IDX_CHUNK, IDX_CHUNK)]],
                out_v.at[pl.ds(c * IDX_CHUNK, IDX_CHUNK)],
                sem)
            for c in range(C_CHUNKS)
        ]
        for h in hs:
            h.wait()

        pltpu.sync_copy(out_v, out_hbm.at[pl.ds(t0 * PQ_M, CODES_BLK)])
        return carry

    lax.fori_loop(0, NBLK, block_body, 0)


def kernel(input_ids, item_codes, centroids):
    ids_flat = input_ids.reshape(N)
    codes_flat = item_codes.reshape((NUM_CODE_WORDS,))
    cent_flat = centroids.reshape(PQ_M * CODEBOOK, SUB)
    mesh = plsc.VectorSubcoreMesh(core_axis_name="c", subcore_axis_name="s")
    f = pl.kernel(
        _sc_body,
        mesh=mesh,
        compiler_params=pltpu.CompilerParams(use_tc_tiling_on_sc=False),
        out_type=jax.ShapeDtypeStruct((N * PQ_M, SUB), jnp.float32),
        scratch_types=[
            pltpu.VMEM((BLK,), jnp.int32),
            pltpu.VMEM((CODES_BLK,), jnp.int32),
            pltpu.VMEM((CODES_BLK,), jnp.int32),
            pltpu.VMEM((CODES_BLK, SUB), jnp.float32),
            pltpu.VMEM_SHARED((PQ_M * CODEBOOK, SUB), jnp.float32),
            pltpu.SemaphoreType.DMA,
        ],
    )
    out = f(ids_flat, codes_flat, cent_flat)
    return out.reshape(B, S, EMB)
